# Initial kernel scaffold; baseline (speedup 1.0000x reference)
import jax
import jax.numpy as jnp
from jax.experimental import pallas as pl


def kernel(embeddings, weights1, weights2, bias1, bias2, sscore_w, sscore_b, pscore, src, rel, dst):
    raise NotImplementedError("write your pallas kernel here")


# speedup vs baseline: 6.9456x; 6.9456x over previous
import dataclasses
import functools

import jax
import jax.numpy as jnp
from jax import lax
from jax.experimental import pallas as pl
from jax.experimental.pallas import tpu as pltpu
from jax.experimental.pallas import tpu_sc as plsc

N = 10000
R = 8
E = 320000
EMB = 128
H = 16
C = 16

NC = 2
NS = 16
NW = NC * NS
EPW = E // NW
K = 80
NCH = EPW // K
L = 16
GR = K // L

_mesh = plsc.VectorSubcoreMesh(core_axis_name="c", subcore_axis_name="s")

_sc_params = pltpu.CompilerParams(
    needs_layout_passes=False, use_tc_tiling_on_sc=False
)


def _s_body(emb_ref, wt_ref, b_ref, out_ref):
    out_ref[...] = (
        jnp.dot(emb_ref[...], wt_ref[...], preferred_element_type=jnp.float32)
        + b_ref[...]
    )


def _xw_body(emb_ref, w1_ref, out_ref):
    out_ref[...] = jnp.dot(
        emb_ref[...], w1_ref[0], preferred_element_type=jnp.float32
    )


def _mid_body(p_ref, b_ref, out_ref):
    out_ref[...] = jnp.maximum(p_ref[0] + p_ref[1] + b_ref[...], 0.0)


def _final_body(q_ref, w2_ref, b_ref, out_ref):
    r = pl.program_id(0)

    @pl.when(r == 0)
    def _():
        out_ref[...] = jnp.broadcast_to(b_ref[...], (N, C))

    h2r = q_ref[0, 0] + q_ref[1, 0]
    out_ref[...] += jnp.dot(h2r, w2_ref[0], preferred_element_type=jnp.float32)


def _sc1_body(s_hbm, ps_hbm, xw_hbm, src_hbm, rel_hbm, dst_hbm,
              values_hbm, h1p_hbm,
              src_v, rel_v, dst_v, xwidx_v, a_v, b_v, xw_v, msg_v,
              vals_v, ps_v, zbuf_v, h1_sh, sem):
    cid = lax.axis_index("c")
    sid = lax.axis_index("s")
    wid = cid * NS + sid
    base = wid * EPW
    zr = 1000

    @pl.loop(0, zr)
    def _(i):
        zbuf_v[i, :] = jnp.zeros((H,), jnp.float32)

    @pl.when(sid < N // zr)
    def _():
        pltpu.sync_copy(zbuf_v, h1_sh.at[pl.ds(sid * zr, zr)])

    pltpu.sync_copy(ps_hbm, ps_v)
    plsc.subcore_barrier()

    iota = lax.broadcasted_iota(jnp.int32, (L,), 0)

    @pl.loop(0, NCH)
    def _chunk(ci):
        off = base + ci * K
        d1 = pltpu.async_copy(src_hbm.at[pl.ds(off, K)], src_v, sem)
        d2 = pltpu.async_copy(rel_hbm.at[pl.ds(off, K)], rel_v, sem)
        d3 = pltpu.async_copy(dst_hbm.at[pl.ds(off, K)], dst_v, sem)
        d1.wait()
        d2.wait()
        d3.wait()

        for g in range(GR):
            sl = pl.ds(g * L, L)
            xwidx_v[sl] = rel_v[sl] * N + dst_v[sl]

        g1 = pltpu.async_copy(s_hbm.at[src_v], a_v, sem)
        g2 = pltpu.async_copy(s_hbm.at[dst_v], b_v, sem)
        g3 = pltpu.async_copy(xw_hbm.at[xwidx_v], xw_v, sem)
        g1.wait()
        g2.wait()
        g3.wait()

        for g in range(GR):
            rows = iota + g * L
            rl = rel_v[pl.ds(g * L, L)]
            acc = jnp.zeros((L,), jnp.float32)
            for h in range(H):
                hv = jnp.full((L,), h, jnp.int32)
                at = plsc.load_gather(a_v, [rows, hv])
                bt = plsc.load_gather(b_v, [rows, hv])
                pt = plsc.load_gather(ps_v, [rl, hv])
                acc = acc + at * pt * bt
            vals_v[pl.ds(g * L, L)] = acc * 0.25

        @pl.loop(0, K)
        def _(e):
            bc = plsc.load_gather(vals_v, [jnp.full((L,), e, jnp.int32)])
            msg_v[e, :] = xw_v[e, :] * bc

        pltpu.sync_copy(msg_v, h1_sh.at[src_v], add=True)
        pltpu.sync_copy(vals_v, values_hbm.at[pl.ds(off, K)])

    plsc.subcore_barrier()

    @pl.when(sid < N // zr)
    def _():
        pltpu.sync_copy(h1_sh.at[pl.ds(sid * zr, zr)],
                        h1p_hbm.at[cid, pl.ds(sid * zr, zr)])


def _sc2_body(h1_hbm, src_hbm, rel_hbm, dst_hbm, values_hbm,
              h2p_hbm,
              src_v, rel_v, dst_v, h2idx_v, hr_v, msg_v, vals_v,
              zbuf_v, h2_sh, sem):
    cid = lax.axis_index("c")
    sid = lax.axis_index("s")
    wid = cid * NS + sid
    base = wid * EPW
    zr = (R * N) // NS
    zb = zbuf_v.shape[0]

    @pl.loop(0, zb)
    def _(i):
        zbuf_v[i, :] = jnp.zeros((H,), jnp.float32)

    for j in range(zr // zb):
        pltpu.sync_copy(zbuf_v, h2_sh.at[pl.ds(sid * zr + j * zb, zb)])
    plsc.subcore_barrier()

    @pl.loop(0, NCH)
    def _chunk(ci):
        off = base + ci * K
        d1 = pltpu.async_copy(src_hbm.at[pl.ds(off, K)], src_v, sem)
        d2 = pltpu.async_copy(rel_hbm.at[pl.ds(off, K)], rel_v, sem)
        d3 = pltpu.async_copy(dst_hbm.at[pl.ds(off, K)], dst_v, sem)
        d4 = pltpu.async_copy(values_hbm.at[pl.ds(off, K)], vals_v, sem)
        d1.wait()
        d2.wait()
        d3.wait()
        d4.wait()

        for g in range(GR):
            sl = pl.ds(g * L, L)
            h2idx_v[sl] = rel_v[sl] * N + src_v[sl]

        pltpu.async_copy(h1_hbm.at[dst_v], hr_v, sem).wait()

        @pl.loop(0, K)
        def _(e):
            bc = plsc.load_gather(vals_v, [jnp.full((L,), e, jnp.int32)])
            msg_v[e, :] = hr_v[e, :] * bc

        pltpu.sync_copy(msg_v, h2_sh.at[h2idx_v], add=True)

    plsc.subcore_barrier()
    pltpu.sync_copy(h2_sh.at[pl.ds(sid * zr, zr)],
                    h2p_hbm.at[cid, pl.ds(sid * zr, zr)])


def kernel(embeddings, weights1, weights2, bias1, bias2, sscore_w, sscore_b,
           pscore, src, rel, dst):
    f32 = jnp.float32
    src = src.astype(jnp.int32)
    rel = rel.astype(jnp.int32)
    dst = dst.astype(jnp.int32)
    sscore_b2 = sscore_b.reshape(1, H)
    bias1_2 = bias1.reshape(1, H)
    bias2_2 = bias2.reshape(1, C)

    s_tab = pl.pallas_call(
        _s_body,
        out_shape=jax.ShapeDtypeStruct((N, H), f32),
    )(embeddings, sscore_w.T, sscore_b2)

    xw_tab = pl.pallas_call(
        _xw_body,
        grid=(R,),
        in_specs=[
            pl.BlockSpec((N, EMB), lambda r: (0, 0)),
            pl.BlockSpec((1, EMB, H), lambda r: (r, 0, 0)),
        ],
        out_specs=pl.BlockSpec((N, H), lambda r: (r, 0)),
        out_shape=jax.ShapeDtypeStruct((R * N, H), f32),
    )(embeddings, weights1)

    sc1 = pl.kernel(
        _sc1_body,
        out_type=[
            jax.ShapeDtypeStruct((E,), f32),
            jax.ShapeDtypeStruct((NC, N, H), f32),
        ],
        mesh=_mesh,
        scratch_types=[
            pltpu.VMEM((K,), jnp.int32),
            pltpu.VMEM((K,), jnp.int32),
            pltpu.VMEM((K,), jnp.int32),
            pltpu.VMEM((K,), jnp.int32),
            pltpu.VMEM((K, H), f32),
            pltpu.VMEM((K, H), f32),
            pltpu.VMEM((K, H), f32),
            pltpu.VMEM((K, H), f32),
            pltpu.VMEM((K,), f32),
            pltpu.VMEM((R, H), f32),
            pltpu.VMEM((1000, H), f32),
            pltpu.VMEM_SHARED((N, H), f32),
            pltpu.SemaphoreType.DMA,
        ],
        compiler_params=_sc_params,
    )
    values, h1p = sc1(s_tab, pscore, xw_tab, src, rel, dst)

    h1 = pl.pallas_call(
        _mid_body,
        out_shape=jax.ShapeDtypeStruct((N, H), f32),
    )(h1p, bias1_2)

    sc2 = pl.kernel(
        _sc2_body,
        out_type=jax.ShapeDtypeStruct((NC, R * N, H), f32),
        mesh=_mesh,
        scratch_types=[
            pltpu.VMEM((K,), jnp.int32),
            pltpu.VMEM((K,), jnp.int32),
            pltpu.VMEM((K,), jnp.int32),
            pltpu.VMEM((K,), jnp.int32),
            pltpu.VMEM((K, H), f32),
            pltpu.VMEM((K, H), f32),
            pltpu.VMEM((K,), f32),
            pltpu.VMEM((1000, H), f32),
            pltpu.VMEM_SHARED((R * N, H), f32),
            pltpu.SemaphoreType.DMA,
        ],
        compiler_params=_sc_params,
    )
    h2p = sc2(h1, src, rel, dst, values)

    out = pl.pallas_call(
        _final_body,
        grid=(R,),
        in_specs=[
            pl.BlockSpec((NC, 1, N, H), lambda r: (0, r, 0, 0)),
            pl.BlockSpec((1, H, C), lambda r: (r, 0, 0)),
            pl.BlockSpec((1, C), lambda r: (0, 0)),
        ],
        out_specs=pl.BlockSpec((N, C), lambda r: (0, 0)),
        out_shape=jax.ShapeDtypeStruct((N, C), f32),
    )(h2p.reshape(NC, R, N, H), weights2, bias2_2)

    return out


# speedup vs baseline: 11.1916x; 1.6113x over previous
import jax
import jax.numpy as jnp
from jax import lax
from jax.experimental import pallas as pl
from jax.experimental.pallas import tpu as pltpu
from jax.experimental.pallas import tpu_sc as plsc

N = 10000
R = 8
E = 320000
EMB = 128
H = 16
C = 16

NC = 2
NS = 16
NW = NC * NS
EPW = E // NW
K = 80
NCH = EPW // K
L = 16
GR = K // L
S_BASE = R * N

_mesh = plsc.VectorSubcoreMesh(core_axis_name="c", subcore_axis_name="s")

_sc_params = pltpu.CompilerParams(
    needs_layout_passes=False, use_tc_tiling_on_sc=False
)


def _xs_body(emb_ref, w_ref, b_ref, out_ref):
    out_ref[...] = (
        jnp.dot(emb_ref[...], w_ref[0], preferred_element_type=jnp.float32)
        + b_ref[0]
    )


def _mid_body(p_ref, b_ref, out_ref):
    out_ref[...] = jnp.maximum(p_ref[0] + p_ref[1] + b_ref[...], 0.0)


def _final_body(q_ref, w2_ref, b_ref, out_ref):
    r = pl.program_id(0)

    @pl.when(r == 0)
    def _():
        out_ref[...] = jnp.broadcast_to(b_ref[...], (N, C))

    h2r = q_ref[0, 0] + q_ref[1, 0]
    out_ref[...] += jnp.dot(h2r, w2_ref[0], preferred_element_type=jnp.float32)


def _sc1_body(xs_hbm, ps_hbm, src_hbm, rel_hbm, dst_hbm,
              values_hbm, h2idx_hbm, h1p_hbm,
              srcA, relA, dstA, vals_all, h2i_all,
              sidx_v, didx_v, xwidx_v, a_v, b_v, xw_v, msg_v,
              ps_v, zbuf_v, h1_sh, sem_gat0, sem_gat1):
    cid = lax.axis_index("c")
    sid = lax.axis_index("s")
    wid = cid * NS + sid
    zr = 1000
    sem_gat = (sem_gat0, sem_gat1)

    pltpu.sync_copy(src_hbm.at[wid], srcA)
    pltpu.sync_copy(rel_hbm.at[wid], relA)
    pltpu.sync_copy(dst_hbm.at[wid], dstA)
    pltpu.sync_copy(ps_hbm, ps_v)

    @pl.loop(0, zr)
    def _(i):
        zbuf_v[i, :] = jnp.zeros((H,), jnp.float32)

    @pl.when(sid < N // zr)
    def _():
        pltpu.sync_copy(zbuf_v, h1_sh.at[pl.ds(sid * zr, zr)])

    plsc.subcore_barrier()

    iota = lax.broadcasted_iota(jnp.int32, (L,), 0)

    def gat_copies(b):
        return (
            pltpu.make_async_copy(xs_hbm.at[sidx_v.at[b]], a_v.at[b],
                                  sem_gat[b]),
            pltpu.make_async_copy(xs_hbm.at[didx_v.at[b]], b_v.at[b],
                                  sem_gat[b]),
            pltpu.make_async_copy(xs_hbm.at[xwidx_v.at[b]], xw_v.at[b],
                                  sem_gat[b]),
        )

    def prefetch(b, ci):
        for g in range(GR):
            sl = pl.ds(g * L, L)
            rl = relA[ci, sl]
            dl = dstA[ci, sl]
            xwidx_v[b, sl] = rl * N + dl
            sidx_v[b, sl] = srcA[ci, sl] + S_BASE
            didx_v[b, sl] = dl + S_BASE
        for c in gat_copies(b):
            c.start()

    def compute(b, ci):
        for c in gat_copies(b):
            c.wait()
        for g in range(GR):
            rows = iota + g * L
            rl = relA[ci, pl.ds(g * L, L)]
            acc = jnp.zeros((L,), jnp.float32)
            for h in range(H):
                hv = jnp.full((L,), h, jnp.int32)
                at = plsc.load_gather(a_v.at[b], [rows, hv])
                bt = plsc.load_gather(b_v.at[b], [rows, hv])
                pt = plsc.load_gather(ps_v, [rl, hv])
                acc = acc + at * pt * bt
            vals_all[ci, pl.ds(g * L, L)] = acc * 0.25
            h2i_all[ci, pl.ds(g * L, L)] = rl * N + srcA[ci, pl.ds(g * L, L)]
            ce = jnp.full((L,), ci, jnp.int32)
            for j in range(L):
                e = g * L + j
                bc = plsc.load_gather(vals_all,
                                      [ce, jnp.full((L,), e, jnp.int32)])
                msg_v[e, :] = xw_v[b, e, :] * bc
        pltpu.sync_copy(msg_v, h1_sh.at[srcA.at[ci]], add=True)

    prefetch(0, 0)

    @pl.loop(0, (NCH - 1) // 2)
    def _pair(k):
        ci = 2 * k
        prefetch(1, ci + 1)
        compute(0, ci)
        prefetch(0, ci + 2)
        compute(1, ci + 1)

    compute(0, NCH - 1)

    plsc.subcore_barrier()

    pltpu.sync_copy(vals_all, values_hbm.at[wid])
    pltpu.sync_copy(h2i_all, h2idx_hbm.at[wid])

    @pl.when(sid < N // zr)
    def _():
        pltpu.sync_copy(h1_sh.at[pl.ds(sid * zr, zr)],
                        h1p_hbm.at[cid, pl.ds(sid * zr, zr)])


def _sc2_body(h1_hbm, dst_hbm, h2idx_hbm, values_hbm,
              h2p_hbm,
              dstA, h2iA, valsA, hr_v, msg_v,
              zbuf_v, h2_sh, sem_gat0, sem_gat1):
    cid = lax.axis_index("c")
    sid = lax.axis_index("s")
    wid = cid * NS + sid
    zr = (R * N) // NS
    zb = zbuf_v.shape[0]
    sem_gat = (sem_gat0, sem_gat1)

    pltpu.sync_copy(dst_hbm.at[wid], dstA)
    pltpu.sync_copy(h2idx_hbm.at[wid], h2iA)
    pltpu.sync_copy(values_hbm.at[wid], valsA)

    @pl.loop(0, zb)
    def _(i):
        zbuf_v[i, :] = jnp.zeros((H,), jnp.float32)

    for j in range(zr // zb):
        pltpu.sync_copy(zbuf_v, h2_sh.at[pl.ds(sid * zr + j * zb, zb)])
    plsc.subcore_barrier()

    def gat_copy(b, ci):
        return pltpu.make_async_copy(h1_hbm.at[dstA.at[ci]], hr_v.at[b],
                                     sem_gat[b])

    def compute(b, ci):
        gat_copy(b, ci).wait()
        ce = jnp.full((L,), ci, jnp.int32)
        for e in range(K):
            bc = plsc.load_gather(valsA, [ce, jnp.full((L,), e, jnp.int32)])
            msg_v[e, :] = hr_v[b, e, :] * bc
        pltpu.sync_copy(msg_v, h2_sh.at[h2iA.at[ci]], add=True)

    gat_copy(0, 0).start()

    @pl.loop(0, (NCH - 1) // 2)
    def _pair(k):
        ci = 2 * k
        gat_copy(1, ci + 1).start()
        compute(0, ci)
        gat_copy(0, ci + 2).start()
        compute(1, ci + 1)

    compute(0, NCH - 1)

    plsc.subcore_barrier()
    pltpu.sync_copy(h2_sh.at[pl.ds(sid * zr, zr)],
                    h2p_hbm.at[cid, pl.ds(sid * zr, zr)])


def kernel(embeddings, weights1, weights2, bias1, bias2, sscore_w, sscore_b,
           pscore, src, rel, dst):
    f32 = jnp.float32
    i32 = jnp.int32
    src3 = src.astype(i32).reshape(NW, NCH, K)
    rel3 = rel.astype(i32).reshape(NW, NCH, K)
    dst3 = dst.astype(i32).reshape(NW, NCH, K)
    bias1_2 = bias1.reshape(1, H)
    bias2_2 = bias2.reshape(1, C)

    w_all = jnp.concatenate([weights1, sscore_w.T[None]], axis=0)
    b_all = jnp.concatenate(
        [jnp.zeros((R, 1, H), f32), sscore_b.reshape(1, 1, H)], axis=0
    )

    xs_tab = pl.pallas_call(
        _xs_body,
        grid=(R + 1,),
        in_specs=[
            pl.BlockSpec((N, EMB), lambda r: (0, 0)),
            pl.BlockSpec((1, EMB, H), lambda r: (r, 0, 0)),
            pl.BlockSpec((1, 1, H), lambda r: (r, 0, 0)),
        ],
        out_specs=pl.BlockSpec((N, H), lambda r: (r, 0)),
        out_shape=jax.ShapeDtypeStruct(((R + 1) * N, H), f32),
    )(embeddings, w_all, b_all)

    sc1 = pl.kernel(
        _sc1_body,
        out_type=[
            jax.ShapeDtypeStruct((NW, NCH, K), f32),
            jax.ShapeDtypeStruct((NW, NCH, K), i32),
            jax.ShapeDtypeStruct((NC, N, H), f32),
        ],
        mesh=_mesh,
        scratch_types=[
            pltpu.VMEM((NCH, K), i32),
            pltpu.VMEM((NCH, K), i32),
            pltpu.VMEM((NCH, K), i32),
            pltpu.VMEM((NCH, K), f32),
            pltpu.VMEM((NCH, K), i32),
            pltpu.VMEM((2, K), i32),
            pltpu.VMEM((2, K), i32),
            pltpu.VMEM((2, K), i32),
            pltpu.VMEM((2, K, H), f32),
            pltpu.VMEM((2, K, H), f32),
            pltpu.VMEM((2, K, H), f32),
            pltpu.VMEM((K, H), f32),
            pltpu.VMEM((R, H), f32),
            pltpu.VMEM((1000, H), f32),
            pltpu.VMEM_SHARED((N, H), f32),
            pltpu.SemaphoreType.DMA,
            pltpu.SemaphoreType.DMA,
        ],
        compiler_params=_sc_params,
    )
    values, h2idx, h1p = sc1(xs_tab, pscore, src3, rel3, dst3)

    h1 = pl.pallas_call(
        _mid_body,
        out_shape=jax.ShapeDtypeStruct((N, H), f32),
    )(h1p, bias1_2)

    sc2 = pl.kernel(
        _sc2_body,
        out_type=jax.ShapeDtypeStruct((NC, R * N, H), f32),
        mesh=_mesh,
        scratch_types=[
            pltpu.VMEM((NCH, K), i32),
            pltpu.VMEM((NCH, K), i32),
            pltpu.VMEM((NCH, K), f32),
            pltpu.VMEM((2, K, H), f32),
            pltpu.VMEM((K, H), f32),
            pltpu.VMEM((1000, H), f32),
            pltpu.VMEM_SHARED((R * N, H), f32),
            pltpu.SemaphoreType.DMA,
            pltpu.SemaphoreType.DMA,
        ],
        compiler_params=_sc_params,
    )
    h2p = sc2(h1, dst3, h2idx, values)

    out = pl.pallas_call(
        _final_body,
        grid=(R,),
        in_specs=[
            pl.BlockSpec((NC, 1, N, H), lambda r: (0, r, 0, 0)),
            pl.BlockSpec((1, H, C), lambda r: (r, 0, 0)),
            pl.BlockSpec((1, C), lambda r: (0, 0)),
        ],
        out_specs=pl.BlockSpec((N, C), lambda r: (0, 0)),
        out_shape=jax.ShapeDtypeStruct((N, C), f32),
    )(h2p.reshape(NC, R, N, H), weights2, bias2_2)

    return out


# speedup vs baseline: 11.5590x; 1.0328x over previous
import jax
import jax.numpy as jnp
from jax import lax
from jax.experimental import pallas as pl
from jax.experimental.pallas import tpu as pltpu
from jax.experimental.pallas import tpu_sc as plsc

N = 10000
R = 8
E = 320000
EMB = 128
H = 16
C = 16

NC = 2
NS = 16
NW = NC * NS
EPW = E // NW
K = 80
NCH = EPW // K
L = 16
GR = K // L
S_BASE = R * N

_mesh = plsc.VectorSubcoreMesh(core_axis_name="c", subcore_axis_name="s")

_sc_params = pltpu.CompilerParams(
    needs_layout_passes=False, use_tc_tiling_on_sc=False
)


def _xs_body(emb_ref, w_ref, b_ref, out_ref):
    out_ref[...] = (
        jnp.dot(emb_ref[...], w_ref[0], preferred_element_type=jnp.float32)
        + b_ref[0]
    )


def _mid_body(p0_ref, p1_ref, b_ref, out_ref):
    out_ref[...] = jnp.maximum(p0_ref[...] + p1_ref[...] + b_ref[...], 0.0)


def _final_body(q_ref, w2_ref, b_ref, out_ref):
    r = pl.program_id(0)

    @pl.when(r == 0)
    def _():
        out_ref[...] = jnp.broadcast_to(b_ref[...], (N, C))

    h2r = q_ref[0, 0] + q_ref[1, 0]
    out_ref[...] += jnp.dot(h2r, w2_ref[0], preferred_element_type=jnp.float32)


def _sc1_body(xs_hbm, ps_hbm, src_hbm, rel_hbm, dst_hbm,
              values_hbm, h2idx_hbm, h1p0_hbm, h1p1_hbm,
              srcA, relA, dstA, vals_all, h2i_all,
              sidx_v, didx_v, xwidx_v, a_v, b_v, xw_v, msg_v,
              ps_v, zbuf_v, h1_sh, sem_gat0, sem_gat1, sem_sc0, sem_sc1):
    cid = lax.axis_index("c")
    sid = lax.axis_index("s")
    wid = cid * NS + sid
    zr = 1000
    sem_gat = (sem_gat0, sem_gat1)
    sem_sc = (sem_sc0, sem_sc1)

    pltpu.sync_copy(src_hbm.at[wid], srcA)
    pltpu.sync_copy(rel_hbm.at[wid], relA)
    pltpu.sync_copy(dst_hbm.at[wid], dstA)
    pltpu.sync_copy(ps_hbm, ps_v)

    @pl.loop(0, zr)
    def _(i):
        zbuf_v[i, :] = jnp.zeros((H,), jnp.float32)

    @pl.when(sid < N // zr)
    def _():
        pltpu.sync_copy(zbuf_v, h1_sh.at[pl.ds(sid * zr, zr)])

    plsc.subcore_barrier()

    iota = lax.broadcasted_iota(jnp.int32, (L,), 0)

    def gat_copies(b):
        return (
            pltpu.make_async_copy(xs_hbm.at[sidx_v.at[b]], a_v.at[b],
                                  sem_gat[b]),
            pltpu.make_async_copy(xs_hbm.at[didx_v.at[b]], b_v.at[b],
                                  sem_gat[b]),
            pltpu.make_async_copy(xs_hbm.at[xwidx_v.at[b]], xw_v.at[b],
                                  sem_gat[b]),
        )

    def prefetch(b, ci):
        for g in range(GR):
            sl = pl.ds(g * L, L)
            rl = relA[ci, sl]
            dl = dstA[ci, sl]
            xwidx_v[b, sl] = rl * N + dl
            sidx_v[b, sl] = srcA[ci, sl] + S_BASE
            didx_v[b, sl] = dl + S_BASE
        for c in gat_copies(b):
            c.start()

    def drain_sc(b):
        pltpu.make_async_copy(msg_v.at[b], h1_sh.at[srcA.at[0]],
                              sem_sc[b]).wait()

    def compute(b, ci, scatter_pending):
        for c in gat_copies(b):
            c.wait()
        for g in range(GR):
            rows = iota + g * L
            rl = relA[ci, pl.ds(g * L, L)]
            acc = jnp.zeros((L,), jnp.float32)
            for h in range(H):
                hv = jnp.full((L,), h, jnp.int32)
                at = plsc.load_gather(a_v.at[b], [rows, hv])
                bt = plsc.load_gather(b_v.at[b], [rows, hv])
                pt = plsc.load_gather(ps_v, [rl, hv])
                acc = acc + at * pt * bt
            vals_all[ci, pl.ds(g * L, L)] = acc * 0.25
            h2i_all[ci, pl.ds(g * L, L)] = rl * N + srcA[ci, pl.ds(g * L, L)]
            ce = jnp.full((L,), ci, jnp.int32)
            for j in range(L):
                e = g * L + j
                bc = plsc.load_gather(vals_all,
                                      [ce, jnp.full((L,), e, jnp.int32)])
                msg_v[b, e, :] = xw_v[b, e, :] * bc
        if scatter_pending:
            drain_sc(1 - b)
        pltpu.make_async_copy(msg_v.at[b], h1_sh.at[srcA.at[ci]],
                              sem_sc[b]).start(add=True)

    prefetch(0, 0)
    prefetch(1, 1)
    compute(0, 0, False)
    prefetch(0, 2)
    compute(1, 1, False)

    @pl.loop(0, (NCH - 3) // 2)
    def _pair(k):
        ci = 2 * k + 2
        prefetch(1, ci + 1)
        compute(0, ci, True)
        prefetch(0, ci + 2)
        compute(1, ci + 1, True)

    compute(0, NCH - 1, True)
    drain_sc(0)

    plsc.subcore_barrier()

    pltpu.sync_copy(vals_all, values_hbm.at[wid])
    pltpu.sync_copy(h2i_all, h2idx_hbm.at[wid])

    @pl.when((sid < N // zr) & (cid == 0))
    def _():
        pltpu.sync_copy(h1_sh.at[pl.ds(sid * zr, zr)],
                        h1p0_hbm.at[pl.ds(sid * zr, zr)])

    @pl.when((sid < N // zr) & (cid == 1))
    def _():
        pltpu.sync_copy(h1_sh.at[pl.ds(sid * zr, zr)],
                        h1p1_hbm.at[pl.ds(sid * zr, zr)])


def _sc2_body(h1_hbm, dst_hbm, h2idx_hbm, values_hbm,
              h2p_hbm,
              dstA, h2iA, valsA, hr0_v, msg_v,
              zbuf_v, h2_sh, sem_gat0, sem_gat1, sem_sc0, sem_sc1):
    cid = lax.axis_index("c")
    sid = lax.axis_index("s")
    wid = cid * NS + sid
    zr = (R * N) // NS
    zb = zbuf_v.shape[0]
    sem_gat = (sem_gat0, sem_gat1)
    sem_sc = (sem_sc0, sem_sc1)

    pltpu.sync_copy(dst_hbm.at[wid], dstA)
    pltpu.sync_copy(h2idx_hbm.at[wid], h2iA)
    pltpu.sync_copy(values_hbm.at[wid], valsA)

    @pl.loop(0, zb)
    def _(i):
        zbuf_v[i, :] = jnp.zeros((H,), jnp.float32)

    for j in range(zr // zb):
        pltpu.sync_copy(zbuf_v, h2_sh.at[pl.ds(sid * zr + j * zb, zb)])
    plsc.subcore_barrier()

    def gat_copies(b, ci):
        return (
            pltpu.make_async_copy(h1_hbm.at[dstA.at[ci]], hr0_v.at[b],
                                  sem_gat[b]),
        )

    def prefetch(b, ci):
        for c in gat_copies(b, ci):
            c.start()

    def drain_sc(b):
        pltpu.make_async_copy(msg_v.at[b], h2_sh.at[h2iA.at[0]],
                              sem_sc[b]).wait()

    def compute(b, ci, scatter_pending):
        for c in gat_copies(b, ci):
            c.wait()
        ce = jnp.full((L,), ci, jnp.int32)
        for e in range(K):
            bc = plsc.load_gather(valsA, [ce, jnp.full((L,), e, jnp.int32)])
            msg_v[b, e, :] = hr0_v[b, e, :] * bc
        if scatter_pending:
            drain_sc(1 - b)
        pltpu.make_async_copy(msg_v.at[b], h2_sh.at[h2iA.at[ci]],
                              sem_sc[b]).start(add=True)

    prefetch(0, 0)
    prefetch(1, 1)
    compute(0, 0, False)
    prefetch(0, 2)
    compute(1, 1, True)

    @pl.loop(0, (NCH - 3) // 2)
    def _pair(k):
        ci = 2 * k + 2
        prefetch(1, ci + 1)
        compute(0, ci, True)
        prefetch(0, ci + 2)
        compute(1, ci + 1, True)

    compute(0, NCH - 1, True)
    drain_sc(0)

    plsc.subcore_barrier()
    pltpu.sync_copy(h2_sh.at[pl.ds(sid * zr, zr)],
                    h2p_hbm.at[cid, pl.ds(sid * zr, zr)])


def kernel(embeddings, weights1, weights2, bias1, bias2, sscore_w, sscore_b,
           pscore, src, rel, dst):
    f32 = jnp.float32
    i32 = jnp.int32
    src3 = src.astype(i32).reshape(NW, NCH, K)
    rel3 = rel.astype(i32).reshape(NW, NCH, K)
    dst3 = dst.astype(i32).reshape(NW, NCH, K)
    bias1_2 = bias1.reshape(1, H)
    bias2_2 = bias2.reshape(1, C)

    w_all = jnp.concatenate([weights1, sscore_w.T[None]], axis=0)
    b_all = jnp.concatenate(
        [jnp.zeros((R, 1, H), f32), sscore_b.reshape(1, 1, H)], axis=0
    )

    xs_tab = pl.pallas_call(
        _xs_body,
        grid=(R + 1,),
        in_specs=[
            pl.BlockSpec((N, EMB), lambda r: (0, 0)),
            pl.BlockSpec((1, EMB, H), lambda r: (r, 0, 0)),
            pl.BlockSpec((1, 1, H), lambda r: (r, 0, 0)),
        ],
        out_specs=pl.BlockSpec((N, H), lambda r: (r, 0)),
        out_shape=jax.ShapeDtypeStruct(((R + 1) * N, H), f32),
    )(embeddings, w_all, b_all)

    sc1 = pl.kernel(
        _sc1_body,
        out_type=[
            jax.ShapeDtypeStruct((NW, NCH, K), f32),
            jax.ShapeDtypeStruct((NW, NCH, K), i32),
            jax.ShapeDtypeStruct((N, H), f32),
            jax.ShapeDtypeStruct((N, H), f32),
        ],
        mesh=_mesh,
        scratch_types=[
            pltpu.VMEM((NCH, K), i32),
            pltpu.VMEM((NCH, K), i32),
            pltpu.VMEM((NCH, K), i32),
            pltpu.VMEM((NCH, K), f32),
            pltpu.VMEM((NCH, K), i32),
            pltpu.VMEM((2, K), i32),
            pltpu.VMEM((2, K), i32),
            pltpu.VMEM((2, K), i32),
            pltpu.VMEM((2, K, H), f32),
            pltpu.VMEM((2, K, H), f32),
            pltpu.VMEM((2, K, H), f32),
            pltpu.VMEM((2, K, H), f32),
            pltpu.VMEM((R, H), f32),
            pltpu.VMEM((1000, H), f32),
            pltpu.VMEM_SHARED((N, H), f32),
            pltpu.SemaphoreType.DMA,
            pltpu.SemaphoreType.DMA,
            pltpu.SemaphoreType.DMA,
            pltpu.SemaphoreType.DMA,
        ],
        compiler_params=_sc_params,
    )
    values, h2idx, h1p0, h1p1 = sc1(xs_tab, pscore, src3, rel3, dst3)

    h1 = pl.pallas_call(
        _mid_body,
        out_shape=jax.ShapeDtypeStruct((N, H), f32),
    )(h1p0, h1p1, bias1_2)

    sc2 = pl.kernel(
        _sc2_body,
        out_type=jax.ShapeDtypeStruct((NC, R * N, H), f32),
        mesh=_mesh,
        scratch_types=[
            pltpu.VMEM((NCH, K), i32),
            pltpu.VMEM((NCH, K), i32),
            pltpu.VMEM((NCH, K), f32),
            pltpu.VMEM((2, K, H), f32),
            pltpu.VMEM((2, K, H), f32),
            pltpu.VMEM((500, H), f32),
            pltpu.VMEM_SHARED((R * N, H), f32),
            pltpu.SemaphoreType.DMA,
            pltpu.SemaphoreType.DMA,
            pltpu.SemaphoreType.DMA,
            pltpu.SemaphoreType.DMA,
        ],
        compiler_params=_sc_params,
    )
    h2p = sc2(h1, dst3, h2idx, values)

    out = pl.pallas_call(
        _final_body,
        grid=(R,),
        in_specs=[
            pl.BlockSpec((NC, 1, N, H), lambda r: (0, r, 0, 0)),
            pl.BlockSpec((1, H, C), lambda r: (r, 0, 0)),
            pl.BlockSpec((1, C), lambda r: (0, 0)),
        ],
        out_specs=pl.BlockSpec((N, C), lambda r: (0, 0)),
        out_shape=jax.ShapeDtypeStruct((N, C), f32),
    )(h2p.reshape(NC, R, N, H), weights2, bias2_2)

    return out
